# baseline (speedup 1.0000x reference)
import functools

import jax
import jax.numpy as jnp
from jax import lax
from jax.experimental import pallas as pl
from jax.experimental.pallas import tpu as pltpu
from jax.experimental.pallas import tpu_sc as plsc

_BATCH = 16384
_FEAT = 1024
_GROUPS = 64
_NC = 2
_NS = 16
_NW = _NC * _NS
_RPW = _BATCH // _NW
_C = 16
_NCHUNK = _RPW // _C
_RING = 3

_mesh = plsc.VectorSubcoreMesh(core_axis_name="c", subcore_axis_name="s")


def _pack_body(mean_ref, std_ref, out_ref):
    r = 1.0 / (std_ref[...] + 1e-5)
    b = mean_ref[...] * r
    rbits = lax.bitcast_convert_type(r.astype(jnp.bfloat16), jnp.uint16)
    bbits = lax.bitcast_convert_type(b.astype(jnp.bfloat16), jnp.uint16)
    w = (rbits.astype(jnp.uint32) << 16) | bbits.astype(jnp.uint32)
    out_ref[...] = w.astype(jnp.int32)


def _pack_table(mean, std):
    return pl.pallas_call(
        _pack_body,
        out_shape=jax.ShapeDtypeStruct((_GROUPS, _FEAT), jnp.int32),
    )(mean, std)


@functools.partial(
    pl.kernel,
    out_type=jax.ShapeDtypeStruct((_BATCH, _FEAT), jnp.float32),
    mesh=_mesh,
    scratch_types=[
        [pltpu.VMEM((_C, _FEAT), jnp.float32) for _ in range(_RING)],
        pltpu.VMEM((_GROUPS, _FEAT), jnp.int32),
        pltpu.VMEM_SHARED((_NS * _RPW,), jnp.int32),
        pltpu.SMEM((_RPW,), jnp.int32),
        [pltpu.SemaphoreType.DMA for _ in range(_RING)],
        [pltpu.SemaphoreType.DMA for _ in range(_RING)],
    ],
    compiler_params=pltpu.CompilerParams(needs_layout_passes=False),
)
def _sc_norm(x_hbm, gidx_hbm, tab_hbm, out_hbm,
             x_v, tab_v, idx_sh, idx_sm, insem, outsem):
    sid = lax.axis_index("s")
    cid = lax.axis_index("c")
    wid = sid * _NC + cid
    base = wid * _RPW

    @pl.when(sid == 0)
    def _stage():
        pltpu.sync_copy(gidx_hbm.at[pl.ds(cid * _NS * _RPW, _NS * _RPW)],
                        idx_sh)

    pltpu.sync_copy(tab_hbm, tab_v)
    plsc.subcore_barrier()
    pltpu.sync_copy(idx_sh.at[pl.ds(sid * _RPW, _RPW)], idx_sm)

    def start_in(c, r):
        pltpu.async_copy(x_hbm.at[pl.ds(base + c * _C, _C)], x_v[r], insem[r])

    def drain_in(r):
        pltpu.make_async_copy(x_hbm.at[pl.ds(0, _C)], x_v[r], insem[r]).wait()

    def wait_out(r):
        pltpu.make_async_copy(x_hbm.at[pl.ds(0, _C)], x_v[r], outsem[r]).wait()

    start_in(0, 0)
    start_in(1, 1)

    hi_mask = jnp.int32(-65536)

    def compute_chunk(c, r):
        @plsc.parallel_loop(0, _C, step=1, unroll=4)
        def row(i):
            g = idx_sm[c * _C + i]
            for j in range(_FEAT // 16):
                sl = pl.ds(j * 16, 16)
                w = tab_v[g, sl]
                rf = plsc.bitcast(w & hi_mask, jnp.float32)
                bf = plsc.bitcast(w << 16, jnp.float32)
                x_v[r][i, sl] = x_v[r][i, sl] * rf - bf

    def outer(k, carry):
        for r in range(_RING):
            c = k * _RING + r
            drain_in(r)
            compute_chunk(c, r)
            pltpu.async_copy(x_v[r], out_hbm.at[pl.ds(base + c * _C, _C)],
                             outsem[r])

            r2 = (r + 2) % _RING

            @pl.when(jnp.logical_and(c >= 1, c + 2 < _NCHUNK))
            def _():
                wait_out(r2)
                start_in(c + 2, r2)

            @pl.when(c == 0)
            def _():
                start_in(2, 2)
        return carry

    lax.fori_loop(0, _NCHUNK // _RING, outer, 0)

    for c in (_NCHUNK - 2, _NCHUNK - 1):
        r = c % _RING
        drain_in(r)
        compute_chunk(c, r)
        pltpu.async_copy(x_v[r], out_hbm.at[pl.ds(base + c * _C, _C)],
                         outsem[r])

    for r in range(_RING):
        wait_out(r)


def kernel(x, group_idx, mean, std):
    tab = _pack_table(mean, std)
    return _sc_norm(x, group_idx.astype(jnp.int32), tab)


# speedup vs baseline: 1.5464x; 1.5464x over previous
import functools

import jax
import jax.numpy as jnp
from jax import lax
from jax.experimental import pallas as pl
from jax.experimental.pallas import tpu as pltpu
from jax.experimental.pallas import tpu_sc as plsc

_BATCH = 16384
_FEAT = 1024
_GROUPS = 64

_SC_ROWS = 4096
_TC_ROWS = _BATCH - _SC_ROWS

_NC = 2
_NS = 16
_NW = _NC * _NS
_RPW = _SC_ROWS // _NW
_C = 16
_NCHUNK = _RPW // _C
_RIN = 2
_ROUT = 2

_mesh = plsc.VectorSubcoreMesh(core_axis_name="c", subcore_axis_name="s")


def _pack_body(mean_ref, std_ref, out_ref):
    r = 1.0 / (std_ref[...] + 1e-5)
    b = mean_ref[...] * r
    rbits = lax.bitcast_convert_type(r.astype(jnp.bfloat16), jnp.uint16)
    bbits = lax.bitcast_convert_type(b.astype(jnp.bfloat16), jnp.uint16)
    w = (rbits.astype(jnp.uint32) << 16) | bbits.astype(jnp.uint32)
    out_ref[...] = w.astype(jnp.int32)


def _pack_table(mean, std):
    return pl.pallas_call(
        _pack_body,
        out_shape=jax.ShapeDtypeStruct((_GROUPS, _FEAT), jnp.int32),
    )(mean, std)


@functools.partial(
    pl.kernel,
    out_type=jax.ShapeDtypeStruct((_SC_ROWS, _FEAT), jnp.float32),
    mesh=_mesh,
    scratch_types=[
        [pltpu.VMEM((_C, _FEAT), jnp.float32) for _ in range(_RIN)],
        [pltpu.VMEM((_C, _FEAT), jnp.int32) for _ in range(_RIN)],
        [pltpu.VMEM((_C, _FEAT), jnp.float32) for _ in range(_ROUT)],
        pltpu.VMEM((_RPW,), jnp.int32),
        [pltpu.SemaphoreType.DMA for _ in range(_RIN)],
        [pltpu.SemaphoreType.DMA for _ in range(_ROUT)],
    ],
    compiler_params=pltpu.CompilerParams(needs_layout_passes=False),
)
def _sc_norm(x_hbm, gidx_hbm, tab_hbm, out_hbm,
             x_v, t_v, y_v, idx_all, insem, outsem):
    sid = lax.axis_index("s")
    wid = sid * _NC + lax.axis_index("c")
    base = wid * _RPW

    pltpu.sync_copy(gidx_hbm.at[pl.ds(base, _RPW)], idx_all)

    def start_in(c, r):
        @pl.when(c < _NCHUNK)
        def _():
            idx_sl = idx_all.at[pl.ds(c * _C, _C)]
            pltpu.async_copy(x_hbm.at[pl.ds(base + c * _C, _C)], x_v[r],
                             insem[r])
            pltpu.async_copy(tab_hbm.at[idx_sl], t_v[r], insem[r])

    def drain_in(r):
        pltpu.make_async_copy(x_hbm.at[pl.ds(0, _C)], x_v[r], insem[r]).wait()
        pltpu.make_async_copy(tab_hbm.at[pl.ds(0, _C)], t_v[r],
                              insem[r]).wait()

    def wait_out(q):
        pltpu.make_async_copy(x_hbm.at[pl.ds(0, _C)], y_v[q],
                              outsem[q]).wait()

    for r in range(_RIN):
        start_in(r, r)

    hi_mask = jnp.int32(-65536)

    def outer(k, carry):
        for r in range(_RIN):
            c = k * _RIN + r
            q = r % _ROUT
            drain_in(r)

            @pl.when(c >= _ROUT)
            def _():
                wait_out(q)

            @plsc.parallel_loop(0, _C, step=1, unroll=4)
            def row(i):
                for j in range(_FEAT // 16):
                    sl = pl.ds(j * 16, 16)
                    w = t_v[r][i, sl]
                    rf = plsc.bitcast(w & hi_mask, jnp.float32)
                    bf = plsc.bitcast(w << 16, jnp.float32)
                    y_v[q][i, sl] = x_v[r][i, sl] * rf - bf

            pltpu.async_copy(y_v[q], out_hbm.at[pl.ds(base + c * _C, _C)],
                             outsem[q])
            start_in(c + _RIN, r)
        return carry

    lax.fori_loop(0, _NCHUNK // _RIN, outer, 0)
    for q in range(_ROUT):
        wait_out(q)


_BR = 1024
_SC_BLOCKS = _SC_ROWS // _BR


def _tc_body(idx_ref, x_ref, mean_ref, std_ref, out_ref):
    g = idx_ref[0, 0, :]
    oh = (g[:, None] == lax.broadcasted_iota(jnp.int32, (_BR, _GROUPS), 1))
    oh = oh.astype(jnp.float32)
    rtab = 1.0 / (std_ref[...] + 1e-5)
    m = jnp.dot(oh, mean_ref[...], preferred_element_type=jnp.float32)
    r = jnp.dot(oh, rtab, preferred_element_type=jnp.float32)
    out_ref[...] = (x_ref[...] - m) * r


def _tc_norm(x, idx3, mean, std):
    grid = _TC_ROWS // _BR
    return pl.pallas_call(
        _tc_body,
        grid=(grid,),
        in_specs=[
            pl.BlockSpec((1, 1, _BR), lambda i: (i + _SC_BLOCKS, 0, 0)),
            pl.BlockSpec((_BR, _FEAT), lambda i: (i + _SC_BLOCKS, 0)),
            pl.BlockSpec((_GROUPS, _FEAT), lambda i: (0, 0)),
            pl.BlockSpec((_GROUPS, _FEAT), lambda i: (0, 0)),
        ],
        out_specs=pl.BlockSpec((_BR, _FEAT), lambda i: (i, 0)),
        out_shape=jax.ShapeDtypeStruct((_TC_ROWS, _FEAT), jnp.float32),
    )(idx3, x, mean, std)


def kernel(x, group_idx, mean, std):
    gi = group_idx.astype(jnp.int32)
    tab = _pack_table(mean, std)
    out_sc = _sc_norm(x, gi, tab)
    idx3 = gi.reshape(_BATCH // _BR, 1, _BR)
    out_tc = _tc_norm(x, idx3, mean, std)
    return jnp.concatenate([out_sc, out_tc], axis=0)
